# baseline (speedup 1.0000x reference)
import jax
import jax.numpy as jnp
from jax.experimental import pallas as pl
from jax.experimental.pallas import tpu as pltpu

D_MODEL = 2048
D_FF = 8192
N_TOKENS = 4096
W_SCALE = 0.02

BF = 256
NBF = D_FF // BF
BN2 = 512
NB2 = N_TOKENS // BN2
BK2 = 1024
NK2 = D_FF // BK2


def _lut_lookup(idx, lut2_ref):
    rows = idx.shape[0]
    m = idx & 127
    tl = jnp.broadcast_to(lut2_ref[0:1, :], (rows, 128))
    th = jnp.broadcast_to(lut2_ref[1:2, :], (rows, 128))
    vlo = jnp.take_along_axis(tl, m, axis=1)
    vhi = jnp.take_along_axis(th, m, axis=1)
    return jnp.where(idx >= 128, vhi, vlo)


def _gate_up_kernel(x_ref, wg_ref, wu_ref, lutg_ref, lutu_ref,
                    srg_ref, sru_ref, slg_ref, slu_ref, srd_ref,
                    out_ref, wgb, wub):
    i = pl.program_id(0)
    slot_r = (i + 1) % 2
    slot_w = i % 2

    x = x_ref[...]
    dims = (((1,), (1,)), ((), ()))
    g = jax.lax.dot_general(x, wgb[slot_r], dims,
                            preferred_element_type=jnp.float32)
    u = jax.lax.dot_general(x, wub[slot_r], dims,
                            preferred_element_type=jnp.float32)
    gs = g * (slg_ref[...] * W_SCALE)
    us = u * (slu_ref[...] * (W_SCALE) * srd_ref[...])
    h = jax.nn.silu(gs) * us
    out_ref[...] = h.astype(jnp.bfloat16)

    wgb[slot_w] = (_lut_lookup(wg_ref[...], lutg_ref)
                   * srg_ref[...]).astype(jnp.bfloat16)
    wub[slot_w] = (_lut_lookup(wu_ref[...], lutu_ref)
                   * sru_ref[...]).astype(jnp.bfloat16)


def _down_kernel(h_ref, wdn_ref, lutd_ref, sld_ref, out_ref, wd_scr):
    n = pl.program_id(0)
    k = pl.program_id(1)

    @pl.when(n == 0)
    def _():
        wd_scr[k] = _lut_lookup(wdn_ref[...], lutd_ref).astype(jnp.bfloat16)

    dims = (((1,), (1,)), ((), ()))
    contrib = jax.lax.dot_general(h_ref[...], wd_scr[k], dims,
                                  preferred_element_type=jnp.float32)

    @pl.when(k == 0)
    def _():
        out_ref[...] = contrib

    @pl.when(jnp.logical_and(k > 0, k < NK2 - 1))
    def _():
        out_ref[...] = out_ref[...] + contrib

    @pl.when(k == NK2 - 1)
    def _():
        out_ref[...] = (out_ref[...] + contrib) * (sld_ref[...] * W_SCALE)


def kernel(x, lut_gate, lut_up, lut_down, walks_gate, walks_up, walks_down,
           sign_l_gate, sign_r_gate, sign_l_up, sign_r_up,
           sign_l_down, sign_r_down):
    x_bf = x.astype(jnp.bfloat16)
    lutg2 = lut_gate.reshape(2, 128)
    lutu2 = lut_up.reshape(2, 128)
    lutd2 = lut_down.reshape(2, 128)
    srg = sign_r_gate.reshape(1, D_MODEL)
    sru = sign_r_up.reshape(1, D_MODEL)
    slg = sign_l_gate.reshape(1, D_FF)
    slu = sign_l_up.reshape(1, D_FF)
    srd = sign_r_down.reshape(1, D_FF)
    sld = sign_l_down.reshape(1, D_MODEL)

    cur = lambda i: (jnp.minimum(i, NBF - 1), 0)
    prev = lambda i: (0, jnp.maximum(i - 1, 0))

    hidden = pl.pallas_call(
        _gate_up_kernel,
        grid=(NBF + 1,),
        in_specs=[
            pl.BlockSpec((N_TOKENS, D_MODEL), lambda i: (0, 0)),
            pl.BlockSpec((BF, D_MODEL), cur),
            pl.BlockSpec((BF, D_MODEL), cur),
            pl.BlockSpec((2, 128), lambda i: (0, 0)),
            pl.BlockSpec((2, 128), lambda i: (0, 0)),
            pl.BlockSpec((1, D_MODEL), lambda i: (0, 0)),
            pl.BlockSpec((1, D_MODEL), lambda i: (0, 0)),
            pl.BlockSpec((1, BF), prev),
            pl.BlockSpec((1, BF), prev),
            pl.BlockSpec((1, BF), prev),
        ],
        out_specs=pl.BlockSpec((N_TOKENS, BF), prev),
        out_shape=jax.ShapeDtypeStruct((N_TOKENS, D_FF), jnp.bfloat16),
        scratch_shapes=[
            pltpu.VMEM((2, BF, D_MODEL), jnp.bfloat16),
            pltpu.VMEM((2, BF, D_MODEL), jnp.bfloat16),
        ],
        compiler_params=pltpu.CompilerParams(
            dimension_semantics=("arbitrary",),
            vmem_limit_bytes=100 * 1024 * 1024,
        ),
    )(x_bf, walks_gate, walks_up, lutg2, lutu2, srg, sru, slg, slu, srd)

    out = pl.pallas_call(
        _down_kernel,
        grid=(NB2, NK2),
        in_specs=[
            pl.BlockSpec((BN2, BK2), lambda n, k: (n, k)),
            pl.BlockSpec((D_MODEL, BK2),
                         lambda n, k: (0, jnp.where(n == 0, k, NK2 - 1))),
            pl.BlockSpec((2, 128), lambda n, k: (0, 0)),
            pl.BlockSpec((1, D_MODEL), lambda n, k: (0, 0)),
        ],
        out_specs=pl.BlockSpec((BN2, D_MODEL), lambda n, k: (n, 0)),
        out_shape=jax.ShapeDtypeStruct((N_TOKENS, D_MODEL), jnp.float32),
        scratch_shapes=[
            pltpu.VMEM((NK2, D_MODEL, BK2), jnp.bfloat16),
        ],
        compiler_params=pltpu.CompilerParams(
            dimension_semantics=("arbitrary", "arbitrary"),
            vmem_limit_bytes=100 * 1024 * 1024,
        ),
    )(hidden, walks_down, lutd2, sld)

    return out


# speedup vs baseline: 1.0425x; 1.0425x over previous
import jax
import jax.numpy as jnp
from jax.experimental import pallas as pl
from jax.experimental.pallas import tpu as pltpu

D_MODEL = 2048
D_FF = 8192
N_TOKENS = 4096
W_SCALE = 0.02

BF = 256
NBF = D_FF // BF
BN2 = 512
NB2 = N_TOKENS // BN2
BK2 = 1024
NK2 = D_FF // BK2


def _lut_lookup(idx, lut2_ref):
    rows = idx.shape[0]
    m = idx & 127
    tl = jnp.broadcast_to(lut2_ref[0:1, :], (rows, 128))
    th = jnp.broadcast_to(lut2_ref[1:2, :], (rows, 128))
    vlo = jnp.take_along_axis(tl, m, axis=1)
    vhi = jnp.take_along_axis(th, m, axis=1)
    return jnp.where(idx >= 128, vhi, vlo)


def _gate_up_kernel(x_ref, wg_ref, wu_ref, lutg_ref, lutu_ref,
                    srg_ref, sru_ref, slg_ref, slu_ref, srd_ref,
                    out_ref, wgb, wub):
    i = pl.program_id(0)
    slot_r = (i + 1) % 2
    slot_w = i % 2

    x = x_ref[...]
    dims = (((1,), (1,)), ((), ()))
    g = jax.lax.dot_general(x, wgb[slot_r], dims,
                            preferred_element_type=jnp.float32)
    u = jax.lax.dot_general(x, wub[slot_r], dims,
                            preferred_element_type=jnp.float32)
    gs = g * (slg_ref[...] * W_SCALE)
    us = u * (slu_ref[...] * (W_SCALE) * srd_ref[...])
    h = jax.nn.silu(gs) * us
    out_ref[...] = h.astype(jnp.bfloat16)

    wgb[slot_w] = (_lut_lookup(wg_ref[...], lutg_ref)
                   * srg_ref[...]).astype(jnp.bfloat16)
    wub[slot_w] = (_lut_lookup(wu_ref[...], lutu_ref)
                   * sru_ref[...]).astype(jnp.bfloat16)


def _down_kernel(h_ref, wdn_ref, lutd_ref, sld_ref, out_ref, wd_scr):
    n = pl.program_id(0)
    k = pl.program_id(1)

    @pl.when(n == 0)
    def _():
        wd_scr[k] = _lut_lookup(wdn_ref[...], lutd_ref).astype(jnp.bfloat16)

    dims = (((1,), (1,)), ((), ()))
    contrib = jax.lax.dot_general(h_ref[...], wd_scr[k], dims,
                                  preferred_element_type=jnp.float32)
    acc = jnp.where(k == 0, contrib, out_ref[...] + contrib)
    scale = jnp.where(k == NK2 - 1, sld_ref[...] * W_SCALE,
                      jnp.ones_like(sld_ref[...]))
    out_ref[...] = acc * scale


def kernel(x, lut_gate, lut_up, lut_down, walks_gate, walks_up, walks_down,
           sign_l_gate, sign_r_gate, sign_l_up, sign_r_up,
           sign_l_down, sign_r_down):
    x_bf = x.astype(jnp.bfloat16)
    lutg2 = lut_gate.reshape(2, 128)
    lutu2 = lut_up.reshape(2, 128)
    lutd2 = lut_down.reshape(2, 128)
    srg = sign_r_gate.reshape(1, D_MODEL)
    sru = sign_r_up.reshape(1, D_MODEL)
    slg = sign_l_gate.reshape(1, D_FF)
    slu = sign_l_up.reshape(1, D_FF)
    srd = sign_r_down.reshape(1, D_FF)
    sld = sign_l_down.reshape(1, D_MODEL)

    cur = lambda i: (jnp.minimum(i, NBF - 1), 0)
    prev = lambda i: (0, jnp.maximum(i - 1, 0))

    hidden = pl.pallas_call(
        _gate_up_kernel,
        grid=(NBF + 1,),
        in_specs=[
            pl.BlockSpec((N_TOKENS, D_MODEL), lambda i: (0, 0)),
            pl.BlockSpec((BF, D_MODEL), cur),
            pl.BlockSpec((BF, D_MODEL), cur),
            pl.BlockSpec((2, 128), lambda i: (0, 0)),
            pl.BlockSpec((2, 128), lambda i: (0, 0)),
            pl.BlockSpec((1, D_MODEL), lambda i: (0, 0)),
            pl.BlockSpec((1, D_MODEL), lambda i: (0, 0)),
            pl.BlockSpec((1, BF), prev),
            pl.BlockSpec((1, BF), prev),
            pl.BlockSpec((1, BF), prev),
        ],
        out_specs=pl.BlockSpec((N_TOKENS, BF), prev),
        out_shape=jax.ShapeDtypeStruct((N_TOKENS, D_FF), jnp.bfloat16),
        scratch_shapes=[
            pltpu.VMEM((2, BF, D_MODEL), jnp.bfloat16),
            pltpu.VMEM((2, BF, D_MODEL), jnp.bfloat16),
        ],
        compiler_params=pltpu.CompilerParams(
            dimension_semantics=("arbitrary",),
            vmem_limit_bytes=100 * 1024 * 1024,
        ),
    )(x_bf, walks_gate, walks_up, lutg2, lutu2, srg, sru, slg, slu, srd)

    out = pl.pallas_call(
        _down_kernel,
        grid=(NB2, NK2),
        in_specs=[
            pl.BlockSpec((BN2, BK2), lambda n, k: (n, k)),
            pl.BlockSpec((D_MODEL, BK2),
                         lambda n, k: (0, jnp.where(n == 0, k, NK2 - 1))),
            pl.BlockSpec((2, 128), lambda n, k: (0, 0)),
            pl.BlockSpec((1, D_MODEL), lambda n, k: (0, 0)),
        ],
        out_specs=pl.BlockSpec((BN2, D_MODEL), lambda n, k: (n, 0)),
        out_shape=jax.ShapeDtypeStruct((N_TOKENS, D_MODEL), jnp.float32),
        scratch_shapes=[
            pltpu.VMEM((NK2, D_MODEL, BK2), jnp.bfloat16),
        ],
        compiler_params=pltpu.CompilerParams(
            dimension_semantics=("arbitrary", "arbitrary"),
            vmem_limit_bytes=100 * 1024 * 1024,
        ),
    )(hidden, walks_down, lutd2, sld)

    return out


# speedup vs baseline: 1.1234x; 1.0776x over previous
import jax
import jax.numpy as jnp
from jax.experimental import pallas as pl
from jax.experimental.pallas import tpu as pltpu

D_MODEL = 2048
D_FF = 8192
N_TOKENS = 4096
W_SCALE = 0.02

BF = 256
NBF = D_FF // BF
BN2 = 512


def _lut_lookup(idx, lut2_ref):
    rows = idx.shape[0]
    m = idx & 127
    tl = jnp.broadcast_to(lut2_ref[0:1, :], (rows, 128))
    th = jnp.broadcast_to(lut2_ref[1:2, :], (rows, 128))
    vlo = jnp.take_along_axis(tl, m, axis=1)
    vhi = jnp.take_along_axis(th, m, axis=1)
    return jnp.where(idx >= 128, vhi, vlo)


def _gate_up_kernel(x_ref, wg_ref, wu_ref, wdn_ref, lutg_ref, lutu_ref,
                    lutd_ref, srg_ref, sru_ref, slg_ref, slu_ref, srd_ref,
                    out_ref, wd_out_ref, wgb, wub):
    i = pl.program_id(0)
    slot_r = (i + 1) % 2
    slot_w = i % 2

    x = x_ref[...]
    dims = (((1,), (1,)), ((), ()))
    g = jax.lax.dot_general(x, wgb[slot_r], dims,
                            preferred_element_type=jnp.float32)
    u = jax.lax.dot_general(x, wub[slot_r], dims,
                            preferred_element_type=jnp.float32)
    gs = g * (slg_ref[...] * W_SCALE)
    us = u * (slu_ref[...] * (W_SCALE) * srd_ref[...])
    h = jax.nn.silu(gs) * us
    out_ref[...] = h.astype(jnp.bfloat16)

    wgb[slot_w] = (_lut_lookup(wg_ref[...], lutg_ref)
                   * srg_ref[...]).astype(jnp.bfloat16)
    wub[slot_w] = (_lut_lookup(wu_ref[...], lutu_ref)
                   * sru_ref[...]).astype(jnp.bfloat16)
    wd_out_ref[...] = _lut_lookup(wdn_ref[...], lutd_ref).astype(jnp.bfloat16)


def _down_kernel(h_ref, wd_ref, sld_ref, out_ref):
    dims = (((1,), (1,)), ((), ()))
    o = jax.lax.dot_general(h_ref[...], wd_ref[...], dims,
                            preferred_element_type=jnp.float32)
    out_ref[...] = o * (sld_ref[...] * W_SCALE)


def kernel(x, lut_gate, lut_up, lut_down, walks_gate, walks_up, walks_down,
           sign_l_gate, sign_r_gate, sign_l_up, sign_r_up,
           sign_l_down, sign_r_down):
    x_bf = x.astype(jnp.bfloat16)
    lutg2 = lut_gate.reshape(2, 128)
    lutu2 = lut_up.reshape(2, 128)
    lutd2 = lut_down.reshape(2, 128)
    srg = sign_r_gate.reshape(1, D_MODEL)
    sru = sign_r_up.reshape(1, D_MODEL)
    slg = sign_l_gate.reshape(1, D_FF)
    slu = sign_l_up.reshape(1, D_FF)
    srd = sign_r_down.reshape(1, D_FF)
    sld = sign_l_down.reshape(1, D_MODEL)

    cur = lambda i: (jnp.minimum(i, NBF - 1), 0)
    curT = lambda i: (0, jnp.minimum(i, NBF - 1))
    prev = lambda i: (0, jnp.maximum(i - 1, 0))

    hidden, wd_bf = pl.pallas_call(
        _gate_up_kernel,
        grid=(NBF + 1,),
        in_specs=[
            pl.BlockSpec((N_TOKENS, D_MODEL), lambda i: (0, 0)),
            pl.BlockSpec((BF, D_MODEL), cur),
            pl.BlockSpec((BF, D_MODEL), cur),
            pl.BlockSpec((D_MODEL, BF), curT),
            pl.BlockSpec((2, 128), lambda i: (0, 0)),
            pl.BlockSpec((2, 128), lambda i: (0, 0)),
            pl.BlockSpec((2, 128), lambda i: (0, 0)),
            pl.BlockSpec((1, D_MODEL), lambda i: (0, 0)),
            pl.BlockSpec((1, D_MODEL), lambda i: (0, 0)),
            pl.BlockSpec((1, BF), prev),
            pl.BlockSpec((1, BF), prev),
            pl.BlockSpec((1, BF), prev),
        ],
        out_specs=[
            pl.BlockSpec((N_TOKENS, BF), prev),
            pl.BlockSpec((D_MODEL, BF), curT),
        ],
        out_shape=[
            jax.ShapeDtypeStruct((N_TOKENS, D_FF), jnp.bfloat16),
            jax.ShapeDtypeStruct((D_MODEL, D_FF), jnp.bfloat16),
        ],
        scratch_shapes=[
            pltpu.VMEM((2, BF, D_MODEL), jnp.bfloat16),
            pltpu.VMEM((2, BF, D_MODEL), jnp.bfloat16),
        ],
        compiler_params=pltpu.CompilerParams(
            dimension_semantics=("arbitrary",),
            vmem_limit_bytes=100 * 1024 * 1024,
        ),
    )(x_bf, walks_gate, walks_up, walks_down, lutg2, lutu2, lutd2,
      srg, sru, slg, slu, srd)

    out = pl.pallas_call(
        _down_kernel,
        grid=(N_TOKENS // BN2,),
        in_specs=[
            pl.BlockSpec((BN2, D_FF), lambda n: (n, 0)),
            pl.BlockSpec((D_MODEL, D_FF), lambda n: (0, 0)),
            pl.BlockSpec((1, D_MODEL), lambda n: (0, 0)),
        ],
        out_specs=pl.BlockSpec((BN2, D_MODEL), lambda n: (n, 0)),
        out_shape=jax.ShapeDtypeStruct((N_TOKENS, D_MODEL), jnp.float32),
        compiler_params=pltpu.CompilerParams(
            dimension_semantics=("arbitrary",),
            vmem_limit_bytes=100 * 1024 * 1024,
        ),
    )(hidden, wd_bf, sld)

    return out
